# baseline (speedup 1.0000x reference)
import functools

import jax
import jax.numpy as jnp
from jax import lax
from jax.experimental import pallas as pl
from jax.experimental.pallas import tpu as pltpu
from jax.experimental.pallas import tpu_sc as plsc

_N = 10000
_E = 320000
_D = 128
_C = 16
_L = 3

_NPAD = 10240
_CH = 128
_NC = 2
_NS = 16
_NW = _NC * _NS
_SLAB = 16
_NSLAB = 5
_CPW = _SLAB * _NSLAB
_EPAD = _NW * _CPW * _CH

_BLK = 1000


def _seg_sum_sc(h, src, dst, zrows):
    mesh = plsc.VectorSubcoreMesh(core_axis_name="c", subcore_axis_name="s")

    @functools.partial(
        pl.kernel,
        out_type=jax.ShapeDtypeStruct((_NC, _NPAD, _D), jnp.float32),
        mesh=mesh,
        scratch_types=[
            pltpu.VMEM((_SLAB, _CH), jnp.int32),
            pltpu.VMEM((_SLAB, _CH), jnp.int32),
            pltpu.VMEM((_SLAB, _CH), jnp.int32),
            pltpu.VMEM((_SLAB, _CH), jnp.int32),
            pltpu.VMEM((_CH, _D), jnp.float32),
            pltpu.VMEM((_CH, _D), jnp.float32),
            pltpu.VMEM_SHARED((_NPAD, _D), jnp.float32),
            pltpu.SemaphoreType.DMA,
            pltpu.SemaphoreType.DMA,
            pltpu.SemaphoreType.DMA,
            pltpu.SemaphoreType.DMA,
            pltpu.SemaphoreType.DMA,
        ],
    )
    def seg_kernel(h_hbm, src_hbm, dst_hbm, z_hbm, out_hbm,
                   src_s0, src_s1, dst_s0, dst_s1, rows0, rows1, acc,
                   semg0, semg1, sems0, sems1, semi):
        cid = lax.axis_index("c")
        sid = lax.axis_index("s")
        wid = sid * _NC + cid

        def gath(ss, j, rows, sem):
            pltpu.async_copy(h_hbm.at[ss.at[j]], rows, sem)

        def wait_gath(ss, j, rows, sem):
            pltpu.make_async_copy(h_hbm.at[ss.at[j]], rows, sem).wait()

        def scat(ds_, j, rows, sem):
            pltpu.async_copy(rows, acc.at[ds_.at[j]], sem, add=True)

        def wait_scat(ds_, j, rows, sem):
            pltpu.make_async_copy(rows, acc.at[ds_.at[j]], sem).wait()

        pltpu.async_copy(src_hbm.at[wid, 0], src_s0, semi)
        pltpu.async_copy(dst_hbm.at[wid, 0], dst_s0, semi)
        pltpu.sync_copy(z_hbm, acc.at[pl.ds(sid * 640, 640)])
        pltpu.make_async_copy(src_hbm.at[wid, 0], src_s0, semi).wait()
        pltpu.make_async_copy(dst_hbm.at[wid, 0], dst_s0, semi).wait()
        plsc.subcore_barrier()
        gath(src_s0, 0, rows0, semg0)
        gath(src_s0, 1, rows1, semg1)

        idx_bufs = [(src_s0, dst_s0), (src_s1, dst_s1)]
        npair = _SLAB // 2
        for s in range(_NSLAB):
            ss, ds_ = idx_bufs[s % 2]
            ns, nd = idx_bufs[(s + 1) % 2]
            if s + 1 < _NSLAB:
                pltpu.async_copy(src_hbm.at[wid, s + 1], ns, semi)
                pltpu.async_copy(dst_hbm.at[wid, s + 1], nd, semi)

            def body(g, carry, ss=ss, ds_=ds_):
                j = 2 * g
                wait_gath(ss, j, rows0, semg0)
                scat(ds_, j, rows0, sems0)
                wait_gath(ss, j + 1, rows1, semg1)
                scat(ds_, j + 1, rows1, sems1)
                wait_scat(ds_, j, rows0, sems0)
                gath(ss, j + 2, rows0, semg0)
                wait_scat(ds_, j + 1, rows1, sems1)
                gath(ss, j + 3, rows1, semg1)
                return carry

            lax.fori_loop(0, npair - 1, body, 0)

            j = _SLAB - 2
            if s + 1 < _NSLAB:
                pltpu.make_async_copy(src_hbm.at[wid, s + 1], ns,
                                      semi).wait()
                pltpu.make_async_copy(dst_hbm.at[wid, s + 1], nd,
                                      semi).wait()
            wait_gath(ss, j, rows0, semg0)
            scat(ds_, j, rows0, sems0)
            wait_gath(ss, j + 1, rows1, semg1)
            scat(ds_, j + 1, rows1, sems1)
            wait_scat(ds_, j, rows0, sems0)
            wait_scat(ds_, j + 1, rows1, sems1)
            if s + 1 < _NSLAB:
                gath(ns, 0, rows0, semg0)
                gath(ns, 1, rows1, semg1)

        plsc.subcore_barrier()

        pltpu.sync_copy(acc.at[pl.ds(sid * 640, 640)],
                        out_hbm.at[cid, pl.ds(sid * 640, 640)])

    return seg_kernel(h, src, dst, zrows)


def _bn_affine(st_ref, g_ref, be_ref):
    mean = st_ref[0:1, :] * (1.0 / _N)
    var = st_ref[1:2, :] * (1.0 / _N) - mean * mean
    scale = g_ref[...] * lax.rsqrt(var + 1e-5)
    shift = be_ref[...] - mean * scale
    return scale, shift


def _layer_body(h_ref, a_ref, w1_ref, b1_ref, g1_ref, be1_ref,
                w2_ref, b2_ref, g2_ref, be2_ref, ho_ref, z1_s, z2_s,
                st1, st2):
    p = pl.program_id(0)
    i = pl.program_id(1)

    @pl.when(jnp.logical_and(p == 0, i == 0))
    def _():
        st1[...] = jnp.zeros_like(st1)
        st2[...] = jnp.zeros_like(st2)

    @pl.when(p == 0)
    def _():
        u = h_ref[...] + a_ref[0] + a_ref[1]
        z = jnp.dot(u, w1_ref[...],
                    preferred_element_type=jnp.float32) + b1_ref[...]
        z1_s[pl.ds(i * _BLK, _BLK), :] = z
        st1[0:1, :] += jnp.sum(z, axis=0, keepdims=True)
        st1[1:2, :] += jnp.sum(z * z, axis=0, keepdims=True)

    @pl.when(p == 1)
    def _():
        scale, shift = _bn_affine(st1, g1_ref, be1_ref)
        a = jnp.maximum(z1_s[pl.ds(i * _BLK, _BLK), :] * scale + shift, 0.0)
        z2 = jnp.dot(a, w2_ref[...],
                     preferred_element_type=jnp.float32) + b2_ref[...]
        z2_s[pl.ds(i * _BLK, _BLK), :] = z2
        st2[0:1, :] += jnp.sum(z2, axis=0, keepdims=True)
        st2[1:2, :] += jnp.sum(z2 * z2, axis=0, keepdims=True)

    @pl.when(p == 2)
    def _():
        scale, shift = _bn_affine(st2, g2_ref, be2_ref)
        ho_ref[...] = jnp.maximum(
            z2_s[pl.ds(i * _BLK, _BLK), :] * scale + shift, 0.0)


def _heads_body(x_ref, h1_ref, h2_ref, h3_ref, fw_ref, fb_ref, o_ref):
    logits = jnp.dot(x_ref[...], fw_ref[0], preferred_element_type=jnp.float32)
    logits += jnp.dot(h1_ref[...], fw_ref[1], preferred_element_type=jnp.float32)
    logits += jnp.dot(h2_ref[...], fw_ref[2], preferred_element_type=jnp.float32)
    logits += jnp.dot(h3_ref[...], fw_ref[3], preferred_element_type=jnp.float32)
    logits += jnp.sum(fb_ref[...], axis=0, keepdims=True)
    m = jnp.max(logits, axis=-1, keepdims=True)
    e = jnp.exp(logits - m)
    o_ref[...] = logits - m - jnp.log(jnp.sum(e, axis=-1, keepdims=True))


def _row_spec():
    return pl.BlockSpec((_BLK, _D), lambda i: (i, 0))


def _full_spec(shape, ng=1):
    nd = len(shape)
    return pl.BlockSpec(shape, lambda *g: (0,) * nd)


def _gin_layer(h, agg2, w1, b1, g1, be1, w2, b2, g2, be2):
    p0row = pl.BlockSpec((_BLK, _D),
                         lambda p, i: (jnp.where(p == 0, i, 0), 0))
    return pl.pallas_call(
        _layer_body,
        grid=(3, _N // _BLK),
        in_specs=[
            p0row,
            pl.BlockSpec((_NC, _BLK, _D),
                         lambda p, i: (0, jnp.where(p == 0, i, 0), 0)),
            _full_spec((_D, _D)),
            _full_spec((1, _D)),
            _full_spec((1, _D)),
            _full_spec((1, _D)),
            _full_spec((_D, _D)),
            _full_spec((1, _D)),
            _full_spec((1, _D)),
            _full_spec((1, _D)),
        ],
        out_specs=pl.BlockSpec((_BLK, _D),
                               lambda p, i: (jnp.where(p == 2, i, 0), 0)),
        out_shape=jax.ShapeDtypeStruct((_N, _D), jnp.float32),
        scratch_shapes=[
            pltpu.VMEM((_N, _D), jnp.float32),
            pltpu.VMEM((_N, _D), jnp.float32),
            pltpu.VMEM((8, _D), jnp.float32),
            pltpu.VMEM((8, _D), jnp.float32),
        ],
    )(h, agg2, w1, b1, g1, be1, w2, b2, g2, be2)


def _heads(x, h1, h2, h3, fw, fb):
    return pl.pallas_call(
        _heads_body,
        grid=(_N // _BLK,),
        in_specs=[
            _row_spec(),
            _row_spec(),
            _row_spec(),
            _row_spec(),
            _full_spec((_L + 1, _D, _C)),
            _full_spec((_L + 1, _C)),
        ],
        out_specs=pl.BlockSpec((_BLK, _C), lambda i: (i, 0)),
        out_shape=jax.ShapeDtypeStruct((_N, _C), jnp.float32),
    )(x, h1, h2, h3, fw, fb)


def kernel(x, edge_index, cw1, cb1, cg1, cbe1, cw2, cb2, bng, bnb, fcw, fcb):
    npad = _EPAD - _E
    pad_i = jnp.arange(npad, dtype=jnp.int32)
    src = jnp.concatenate([edge_index[0], (pad_i * 37) % _N]).reshape(
        _NW, _NSLAB, _SLAB, _CH)
    dst = jnp.concatenate([edge_index[1], _N + pad_i % (_NPAD - _N)]).reshape(
        _NW, _NSLAB, _SLAB, _CH)
    zrows = jnp.zeros((640, _D), jnp.float32)

    h = x
    outs = [x]
    for l in range(_L):
        agg2 = _seg_sum_sc(h, src, dst, zrows)
        h = _gin_layer(h, agg2, cw1[l], cb1[l].reshape(1, _D),
                       cg1[l].reshape(1, _D), cbe1[l].reshape(1, _D),
                       cw2[l], cb2[l].reshape(1, _D),
                       bng[l].reshape(1, _D), bnb[l].reshape(1, _D))
        outs.append(h)

    out = _heads(outs[0], outs[1], outs[2], outs[3], fcw, fcb)
    return (out, 0)


# speedup vs baseline: 1.2540x; 1.2540x over previous
import functools

import jax
import jax.numpy as jnp
from jax import lax
from jax.experimental import pallas as pl
from jax.experimental.pallas import tpu as pltpu
from jax.experimental.pallas import tpu_sc as plsc

_N = 10000
_E = 320000
_D = 128
_C = 16
_L = 3

_NPAD = 10240
_CH = 128
_NC = 2
_NS = 16
_NW = _NC * _NS
_SLAB = 16
_NSLAB = 5
_CPW = _SLAB * _NSLAB
_EPAD = _NW * _CPW * _CH

_BLK = 1000


def _seg_sum_sc(h, src, dst, zrows):
    mesh = plsc.VectorSubcoreMesh(core_axis_name="c", subcore_axis_name="s")

    @functools.partial(
        pl.kernel,
        out_type=jax.ShapeDtypeStruct((_NC, _NPAD, _D), jnp.float32),
        mesh=mesh,
        scratch_types=[
            pltpu.VMEM((_SLAB, _CH), jnp.int32),
            pltpu.VMEM((_SLAB, _CH), jnp.int32),
            pltpu.VMEM((_SLAB, _CH), jnp.int32),
            pltpu.VMEM((_SLAB, _CH), jnp.int32),
            pltpu.VMEM((_CH, _D), jnp.float32),
            pltpu.VMEM((_CH, _D), jnp.float32),
            pltpu.VMEM_SHARED((_NPAD, _D), jnp.float32),
            pltpu.SemaphoreType.DMA,
            pltpu.SemaphoreType.DMA,
            pltpu.SemaphoreType.DMA,
        ],
    )
    def seg_kernel(h_hbm, src_hbm, dst_hbm, z_hbm, out_hbm,
                   src_s0, src_s1, dst_s0, dst_s1, rows0, rows1, acc,
                   semg0, semg1, semi):
        cid = lax.axis_index("c")
        sid = lax.axis_index("s")
        wid = sid * _NC + cid

        def gath(ss, j, rows, sem):
            pltpu.async_copy(h_hbm.at[ss.at[j]], rows, sem)

        def wait_gath(ss, j, rows, sem):
            pltpu.make_async_copy(h_hbm.at[ss.at[j]], rows, sem).wait()

        def scat(ds_, j, rows):
            pltpu.sync_copy(rows, acc.at[ds_.at[j]], add=True)

        pltpu.async_copy(src_hbm.at[wid, 0], src_s0, semi)
        pltpu.async_copy(dst_hbm.at[wid, 0], dst_s0, semi)
        pltpu.sync_copy(z_hbm, acc.at[pl.ds(sid * 640, 640)])
        pltpu.make_async_copy(src_hbm.at[wid, 0], src_s0, semi).wait()
        pltpu.make_async_copy(dst_hbm.at[wid, 0], dst_s0, semi).wait()
        plsc.subcore_barrier()
        gath(src_s0, 0, rows0, semg0)

        idx_bufs = [(src_s0, dst_s0), (src_s1, dst_s1)]
        npair = _SLAB // 2
        for s in range(_NSLAB):
            ss, ds_ = idx_bufs[s % 2]
            ns, nd = idx_bufs[(s + 1) % 2]
            if s + 1 < _NSLAB:
                pltpu.async_copy(src_hbm.at[wid, s + 1], ns, semi)
                pltpu.async_copy(dst_hbm.at[wid, s + 1], nd, semi)

            def body(g, carry, ss=ss, ds_=ds_):
                j = 2 * g
                gath(ss, j + 1, rows1, semg1)
                wait_gath(ss, j, rows0, semg0)
                scat(ds_, j, rows0)
                gath(ss, j + 2, rows0, semg0)
                wait_gath(ss, j + 1, rows1, semg1)
                scat(ds_, j + 1, rows1)
                return carry

            lax.fori_loop(0, npair - 1, body, 0)

            j = _SLAB - 2
            gath(ss, j + 1, rows1, semg1)
            if s + 1 < _NSLAB:
                pltpu.make_async_copy(src_hbm.at[wid, s + 1], ns,
                                      semi).wait()
                pltpu.make_async_copy(dst_hbm.at[wid, s + 1], nd,
                                      semi).wait()
            wait_gath(ss, j, rows0, semg0)
            scat(ds_, j, rows0)
            if s + 1 < _NSLAB:
                gath(ns, 0, rows0, semg0)
            wait_gath(ss, j + 1, rows1, semg1)
            scat(ds_, j + 1, rows1)

        plsc.subcore_barrier()

        pltpu.sync_copy(acc.at[pl.ds(sid * 640, 640)],
                        out_hbm.at[cid, pl.ds(sid * 640, 640)])

    return seg_kernel(h, src, dst, zrows)


def _bn_affine(st_ref, g_ref, be_ref):
    mean = st_ref[0:1, :] * (1.0 / _N)
    var = st_ref[1:2, :] * (1.0 / _N) - mean * mean
    scale = g_ref[...] * lax.rsqrt(var + 1e-5)
    shift = be_ref[...] - mean * scale
    return scale, shift


def _layer_body(h_ref, a_ref, w1_ref, b1_ref, g1_ref, be1_ref,
                w2_ref, b2_ref, g2_ref, be2_ref, ho_ref, z1_s, z2_s,
                st1, st2):
    p = pl.program_id(0)
    i = pl.program_id(1)

    @pl.when(jnp.logical_and(p == 0, i == 0))
    def _():
        st1[...] = jnp.zeros_like(st1)
        st2[...] = jnp.zeros_like(st2)

    @pl.when(p == 0)
    def _():
        u = h_ref[...] + a_ref[0] + a_ref[1]
        z = jnp.dot(u, w1_ref[...],
                    preferred_element_type=jnp.float32) + b1_ref[...]
        z1_s[pl.ds(i * _BLK, _BLK), :] = z
        st1[0:1, :] += jnp.sum(z, axis=0, keepdims=True)
        st1[1:2, :] += jnp.sum(z * z, axis=0, keepdims=True)

    @pl.when(p == 1)
    def _():
        scale, shift = _bn_affine(st1, g1_ref, be1_ref)
        a = jnp.maximum(z1_s[pl.ds(i * _BLK, _BLK), :] * scale + shift, 0.0)
        z2 = jnp.dot(a, w2_ref[...],
                     preferred_element_type=jnp.float32) + b2_ref[...]
        z2_s[pl.ds(i * _BLK, _BLK), :] = z2
        st2[0:1, :] += jnp.sum(z2, axis=0, keepdims=True)
        st2[1:2, :] += jnp.sum(z2 * z2, axis=0, keepdims=True)

    @pl.when(p == 2)
    def _():
        scale, shift = _bn_affine(st2, g2_ref, be2_ref)
        ho_ref[...] = jnp.maximum(
            z2_s[pl.ds(i * _BLK, _BLK), :] * scale + shift, 0.0)


def _heads_body(x_ref, h1_ref, h2_ref, h3_ref, fw_ref, fb_ref, o_ref):
    logits = jnp.dot(x_ref[...], fw_ref[0], preferred_element_type=jnp.float32)
    logits += jnp.dot(h1_ref[...], fw_ref[1], preferred_element_type=jnp.float32)
    logits += jnp.dot(h2_ref[...], fw_ref[2], preferred_element_type=jnp.float32)
    logits += jnp.dot(h3_ref[...], fw_ref[3], preferred_element_type=jnp.float32)
    logits += jnp.sum(fb_ref[...], axis=0, keepdims=True)
    m = jnp.max(logits, axis=-1, keepdims=True)
    e = jnp.exp(logits - m)
    o_ref[...] = logits - m - jnp.log(jnp.sum(e, axis=-1, keepdims=True))


def _row_spec():
    return pl.BlockSpec((_BLK, _D), lambda i: (i, 0))


def _full_spec(shape, ng=1):
    nd = len(shape)
    return pl.BlockSpec(shape, lambda *g: (0,) * nd)


def _gin_layer(h, agg2, w1, b1, g1, be1, w2, b2, g2, be2):
    p0row = pl.BlockSpec((_BLK, _D),
                         lambda p, i: (jnp.where(p == 0, i, 0), 0))
    return pl.pallas_call(
        _layer_body,
        grid=(3, _N // _BLK),
        in_specs=[
            p0row,
            pl.BlockSpec((_NC, _BLK, _D),
                         lambda p, i: (0, jnp.where(p == 0, i, 0), 0)),
            _full_spec((_D, _D)),
            _full_spec((1, _D)),
            _full_spec((1, _D)),
            _full_spec((1, _D)),
            _full_spec((_D, _D)),
            _full_spec((1, _D)),
            _full_spec((1, _D)),
            _full_spec((1, _D)),
        ],
        out_specs=pl.BlockSpec((_BLK, _D),
                               lambda p, i: (jnp.where(p == 2, i, 0), 0)),
        out_shape=jax.ShapeDtypeStruct((_N, _D), jnp.float32),
        scratch_shapes=[
            pltpu.VMEM((_N, _D), jnp.float32),
            pltpu.VMEM((_N, _D), jnp.float32),
            pltpu.VMEM((8, _D), jnp.float32),
            pltpu.VMEM((8, _D), jnp.float32),
        ],
    )(h, agg2, w1, b1, g1, be1, w2, b2, g2, be2)


def _heads(x, h1, h2, h3, fw, fb):
    return pl.pallas_call(
        _heads_body,
        grid=(_N // _BLK,),
        in_specs=[
            _row_spec(),
            _row_spec(),
            _row_spec(),
            _row_spec(),
            _full_spec((_L + 1, _D, _C)),
            _full_spec((_L + 1, _C)),
        ],
        out_specs=pl.BlockSpec((_BLK, _C), lambda i: (i, 0)),
        out_shape=jax.ShapeDtypeStruct((_N, _C), jnp.float32),
    )(x, h1, h2, h3, fw, fb)


def kernel(x, edge_index, cw1, cb1, cg1, cbe1, cw2, cb2, bng, bnb, fcw, fcb):
    npad = _EPAD - _E
    pad_i = jnp.arange(npad, dtype=jnp.int32)
    src = jnp.concatenate([edge_index[0], (pad_i * 37) % _N]).reshape(
        _NW, _NSLAB, _SLAB, _CH)
    dst = jnp.concatenate([edge_index[1], _N + pad_i % (_NPAD - _N)]).reshape(
        _NW, _NSLAB, _SLAB, _CH)
    zrows = jnp.zeros((640, _D), jnp.float32)

    h = x
    outs = [x]
    for l in range(_L):
        agg2 = _seg_sum_sc(h, src, dst, zrows)
        h = _gin_layer(h, agg2, cw1[l], cb1[l].reshape(1, _D),
                       cg1[l].reshape(1, _D), cbe1[l].reshape(1, _D),
                       cw2[l], cb2[l].reshape(1, _D),
                       bng[l].reshape(1, _D), bnb[l].reshape(1, _D))
        outs.append(h)

    out = _heads(outs[0], outs[1], outs[2], outs[3], fcw, fcb)
    return (out, 0)


# speedup vs baseline: 1.3276x; 1.0587x over previous
import functools

import jax
import jax.numpy as jnp
from jax import lax
from jax.experimental import pallas as pl
from jax.experimental.pallas import tpu as pltpu
from jax.experimental.pallas import tpu_sc as plsc

_N = 10000
_E = 320000
_D = 128
_C = 16
_L = 3

_NPAD = 10240
_CH = 128
_NC = 2
_NS = 16
_NW = _NC * _NS
_SLAB = 16
_NSLAB = 5
_CPW = _SLAB * _NSLAB
_EPAD = _NW * _CPW * _CH

_BLK = 2000


def _seg_sum_sc(h, src, dst, zrows):
    mesh = plsc.VectorSubcoreMesh(core_axis_name="c", subcore_axis_name="s")

    @functools.partial(
        pl.kernel,
        out_type=jax.ShapeDtypeStruct((_NC, _NPAD, _D), jnp.float32),
        mesh=mesh,
        scratch_types=[
            pltpu.VMEM((_SLAB, _CH), jnp.int32),
            pltpu.VMEM((_SLAB, _CH), jnp.int32),
            pltpu.VMEM((_SLAB, _CH), jnp.int32),
            pltpu.VMEM((_SLAB, _CH), jnp.int32),
            pltpu.VMEM((_CH, _D), jnp.float32),
            pltpu.VMEM((_CH, _D), jnp.float32),
            pltpu.VMEM_SHARED((_NPAD, _D), jnp.float32),
            pltpu.SemaphoreType.DMA,
            pltpu.SemaphoreType.DMA,
            pltpu.SemaphoreType.DMA,
        ],
    )
    def seg_kernel(h_hbm, src_hbm, dst_hbm, z_hbm, out_hbm,
                   src_s0, src_s1, dst_s0, dst_s1, rows0, rows1, acc,
                   semg0, semg1, semi):
        cid = lax.axis_index("c")
        sid = lax.axis_index("s")
        wid = sid * _NC + cid

        def gath(ss, j, rows, sem):
            pltpu.async_copy(h_hbm.at[ss.at[j]], rows, sem)

        def wait_gath(ss, j, rows, sem):
            pltpu.make_async_copy(h_hbm.at[ss.at[j]], rows, sem).wait()

        def scat(ds_, j, rows):
            pltpu.sync_copy(rows, acc.at[ds_.at[j]], add=True)

        pltpu.async_copy(src_hbm.at[wid, 0], src_s0, semi)
        pltpu.async_copy(dst_hbm.at[wid, 0], dst_s0, semi)
        pltpu.sync_copy(z_hbm, acc.at[pl.ds(sid * 640, 640)])
        pltpu.make_async_copy(src_hbm.at[wid, 0], src_s0, semi).wait()
        pltpu.make_async_copy(dst_hbm.at[wid, 0], dst_s0, semi).wait()
        plsc.subcore_barrier()
        gath(src_s0, 0, rows0, semg0)

        idx_bufs = [(src_s0, dst_s0), (src_s1, dst_s1)]
        npair = _SLAB // 2
        for s in range(_NSLAB):
            ss, ds_ = idx_bufs[s % 2]
            ns, nd = idx_bufs[(s + 1) % 2]
            if s + 1 < _NSLAB:
                pltpu.async_copy(src_hbm.at[wid, s + 1], ns, semi)
                pltpu.async_copy(dst_hbm.at[wid, s + 1], nd, semi)

            def body(g, carry, ss=ss, ds_=ds_):
                j = 2 * g
                gath(ss, j + 1, rows1, semg1)
                wait_gath(ss, j, rows0, semg0)
                scat(ds_, j, rows0)
                gath(ss, j + 2, rows0, semg0)
                wait_gath(ss, j + 1, rows1, semg1)
                scat(ds_, j + 1, rows1)
                return carry

            lax.fori_loop(0, npair - 1, body, 0)

            j = _SLAB - 2
            gath(ss, j + 1, rows1, semg1)
            if s + 1 < _NSLAB:
                pltpu.make_async_copy(src_hbm.at[wid, s + 1], ns,
                                      semi).wait()
                pltpu.make_async_copy(dst_hbm.at[wid, s + 1], nd,
                                      semi).wait()
            wait_gath(ss, j, rows0, semg0)
            scat(ds_, j, rows0)
            if s + 1 < _NSLAB:
                gath(ns, 0, rows0, semg0)
            wait_gath(ss, j + 1, rows1, semg1)
            scat(ds_, j + 1, rows1)

        plsc.subcore_barrier()

        pltpu.sync_copy(acc.at[pl.ds(sid * 640, 640)],
                        out_hbm.at[cid, pl.ds(sid * 640, 640)])

    return seg_kernel(h, src, dst, zrows)


def _bn_affine(st_ref, g_ref, be_ref):
    mean = st_ref[0:1, :] * (1.0 / _N)
    var = st_ref[1:2, :] * (1.0 / _N) - mean * mean
    scale = g_ref[...] * lax.rsqrt(var + 1e-5)
    shift = be_ref[...] - mean * scale
    return scale, shift


def _layer_body(h_ref, a_ref, w1_ref, b1_ref, g1_ref, be1_ref,
                w2_ref, b2_ref, g2_ref, be2_ref, ho_ref, z1_s, z2_s,
                st1, st2):
    p = pl.program_id(0)
    i = pl.program_id(1)

    @pl.when(jnp.logical_and(p == 0, i == 0))
    def _():
        st1[...] = jnp.zeros_like(st1)
        st2[...] = jnp.zeros_like(st2)

    @pl.when(p == 0)
    def _():
        u = h_ref[...] + a_ref[0] + a_ref[1]
        z = jnp.dot(u, w1_ref[...],
                    preferred_element_type=jnp.float32) + b1_ref[...]
        z1_s[pl.ds(i * _BLK, _BLK), :] = z
        st1[0:1, :] += jnp.sum(z, axis=0, keepdims=True)
        st1[1:2, :] += jnp.sum(z * z, axis=0, keepdims=True)

    @pl.when(p == 1)
    def _():
        scale, shift = _bn_affine(st1, g1_ref, be1_ref)
        a = jnp.maximum(z1_s[pl.ds(i * _BLK, _BLK), :] * scale + shift, 0.0)
        z2 = jnp.dot(a, w2_ref[...],
                     preferred_element_type=jnp.float32) + b2_ref[...]
        z2_s[pl.ds(i * _BLK, _BLK), :] = z2
        st2[0:1, :] += jnp.sum(z2, axis=0, keepdims=True)
        st2[1:2, :] += jnp.sum(z2 * z2, axis=0, keepdims=True)

    @pl.when(p == 2)
    def _():
        scale, shift = _bn_affine(st2, g2_ref, be2_ref)
        ho_ref[...] = jnp.maximum(
            z2_s[pl.ds(i * _BLK, _BLK), :] * scale + shift, 0.0)


def _heads_body(x_ref, h1_ref, h2_ref, h3_ref, fw_ref, fb_ref, o_ref):
    logits = jnp.dot(x_ref[...], fw_ref[0], preferred_element_type=jnp.float32)
    logits += jnp.dot(h1_ref[...], fw_ref[1], preferred_element_type=jnp.float32)
    logits += jnp.dot(h2_ref[...], fw_ref[2], preferred_element_type=jnp.float32)
    logits += jnp.dot(h3_ref[...], fw_ref[3], preferred_element_type=jnp.float32)
    logits += jnp.sum(fb_ref[...], axis=0, keepdims=True)
    m = jnp.max(logits, axis=-1, keepdims=True)
    e = jnp.exp(logits - m)
    o_ref[...] = logits - m - jnp.log(jnp.sum(e, axis=-1, keepdims=True))


def _row_spec():
    return pl.BlockSpec((_BLK, _D), lambda i: (i, 0))


def _full_spec(shape, ng=1):
    nd = len(shape)
    return pl.BlockSpec(shape, lambda *g: (0,) * nd)


def _gin_layer(h, agg2, w1, b1, g1, be1, w2, b2, g2, be2):
    p0row = pl.BlockSpec((_BLK, _D),
                         lambda p, i: (jnp.where(p == 0, i, 0), 0))
    return pl.pallas_call(
        _layer_body,
        grid=(3, _N // _BLK),
        in_specs=[
            p0row,
            pl.BlockSpec((_NC, _BLK, _D),
                         lambda p, i: (0, jnp.where(p == 0, i, 0), 0)),
            _full_spec((_D, _D)),
            _full_spec((1, _D)),
            _full_spec((1, _D)),
            _full_spec((1, _D)),
            _full_spec((_D, _D)),
            _full_spec((1, _D)),
            _full_spec((1, _D)),
            _full_spec((1, _D)),
        ],
        out_specs=pl.BlockSpec((_BLK, _D),
                               lambda p, i: (jnp.where(p == 2, i, 0), 0)),
        out_shape=jax.ShapeDtypeStruct((_N, _D), jnp.float32),
        scratch_shapes=[
            pltpu.VMEM((_N, _D), jnp.float32),
            pltpu.VMEM((_N, _D), jnp.float32),
            pltpu.VMEM((8, _D), jnp.float32),
            pltpu.VMEM((8, _D), jnp.float32),
        ],
    )(h, agg2, w1, b1, g1, be1, w2, b2, g2, be2)


def _heads(x, h1, h2, h3, fw, fb):
    return pl.pallas_call(
        _heads_body,
        grid=(_N // _BLK,),
        in_specs=[
            _row_spec(),
            _row_spec(),
            _row_spec(),
            _row_spec(),
            _full_spec((_L + 1, _D, _C)),
            _full_spec((_L + 1, _C)),
        ],
        out_specs=pl.BlockSpec((_BLK, _C), lambda i: (i, 0)),
        out_shape=jax.ShapeDtypeStruct((_N, _C), jnp.float32),
    )(x, h1, h2, h3, fw, fb)


def kernel(x, edge_index, cw1, cb1, cg1, cbe1, cw2, cb2, bng, bnb, fcw, fcb):
    npad = _EPAD - _E
    pad_i = jnp.arange(npad, dtype=jnp.int32)
    src = jnp.concatenate([edge_index[0], (pad_i * 37) % _N]).reshape(
        _NW, _NSLAB, _SLAB, _CH)
    dst = jnp.concatenate([edge_index[1], _N + pad_i % (_NPAD - _N)]).reshape(
        _NW, _NSLAB, _SLAB, _CH)
    zrows = jnp.zeros((640, _D), jnp.float32)

    h = x
    outs = [x]
    for l in range(_L):
        agg2 = _seg_sum_sc(h, src, dst, zrows)
        h = _gin_layer(h, agg2, cw1[l], cb1[l].reshape(1, _D),
                       cg1[l].reshape(1, _D), cbe1[l].reshape(1, _D),
                       cw2[l], cb2[l].reshape(1, _D),
                       bng[l].reshape(1, _D), bnb[l].reshape(1, _D))
        outs.append(h)

    out = _heads(outs[0], outs[1], outs[2], outs[3], fcw, fcb)
    return (out, 0)


# speedup vs baseline: 1.3315x; 1.0029x over previous
import functools

import jax
import jax.numpy as jnp
from jax import lax
from jax.experimental import pallas as pl
from jax.experimental.pallas import tpu as pltpu
from jax.experimental.pallas import tpu_sc as plsc

_N = 10000
_E = 320000
_D = 128
_C = 16
_L = 3

_NPAD = 10240
_CH = 128
_NC = 2
_NS = 16
_NW = _NC * _NS
_SLAB = 16
_NSLAB = 5
_CPW = _SLAB * _NSLAB
_EPAD = _NW * _CPW * _CH

_BLK = 5000


def _seg_sum_sc(h, src, dst, zrows):
    mesh = plsc.VectorSubcoreMesh(core_axis_name="c", subcore_axis_name="s")

    @functools.partial(
        pl.kernel,
        out_type=jax.ShapeDtypeStruct((_NC, _NPAD, _D), jnp.float32),
        mesh=mesh,
        scratch_types=[
            pltpu.VMEM((_SLAB, _CH), jnp.int32),
            pltpu.VMEM((_SLAB, _CH), jnp.int32),
            pltpu.VMEM((_SLAB, _CH), jnp.int32),
            pltpu.VMEM((_SLAB, _CH), jnp.int32),
            pltpu.VMEM((_CH, _D), jnp.float32),
            pltpu.VMEM((_CH, _D), jnp.float32),
            pltpu.VMEM_SHARED((_NPAD, _D), jnp.float32),
            pltpu.SemaphoreType.DMA,
            pltpu.SemaphoreType.DMA,
            pltpu.SemaphoreType.DMA,
        ],
    )
    def seg_kernel(h_hbm, src_hbm, dst_hbm, z_hbm, out_hbm,
                   src_s0, src_s1, dst_s0, dst_s1, rows0, rows1, acc,
                   semg0, semg1, semi):
        cid = lax.axis_index("c")
        sid = lax.axis_index("s")
        wid = sid * _NC + cid

        def gath(ss, j, rows, sem):
            pltpu.async_copy(h_hbm.at[ss.at[j]], rows, sem)

        def wait_gath(ss, j, rows, sem):
            pltpu.make_async_copy(h_hbm.at[ss.at[j]], rows, sem).wait()

        def scat(ds_, j, rows):
            pltpu.sync_copy(rows, acc.at[ds_.at[j]], add=True)

        pltpu.async_copy(src_hbm.at[wid, 0], src_s0, semi)
        pltpu.async_copy(dst_hbm.at[wid, 0], dst_s0, semi)
        pltpu.sync_copy(z_hbm, acc.at[pl.ds(sid * 640, 640)])
        pltpu.make_async_copy(src_hbm.at[wid, 0], src_s0, semi).wait()
        pltpu.make_async_copy(dst_hbm.at[wid, 0], dst_s0, semi).wait()
        plsc.subcore_barrier()
        gath(src_s0, 0, rows0, semg0)

        idx_bufs = [(src_s0, dst_s0), (src_s1, dst_s1)]
        npair = _SLAB // 2
        for s in range(_NSLAB):
            ss, ds_ = idx_bufs[s % 2]
            ns, nd = idx_bufs[(s + 1) % 2]
            if s + 1 < _NSLAB:
                pltpu.async_copy(src_hbm.at[wid, s + 1], ns, semi)
                pltpu.async_copy(dst_hbm.at[wid, s + 1], nd, semi)

            def body(g, carry, ss=ss, ds_=ds_):
                j = 2 * g
                gath(ss, j + 1, rows1, semg1)
                wait_gath(ss, j, rows0, semg0)
                scat(ds_, j, rows0)
                gath(ss, j + 2, rows0, semg0)
                wait_gath(ss, j + 1, rows1, semg1)
                scat(ds_, j + 1, rows1)
                return carry

            lax.fori_loop(0, npair - 1, body, 0)

            j = _SLAB - 2
            gath(ss, j + 1, rows1, semg1)
            if s + 1 < _NSLAB:
                pltpu.make_async_copy(src_hbm.at[wid, s + 1], ns,
                                      semi).wait()
                pltpu.make_async_copy(dst_hbm.at[wid, s + 1], nd,
                                      semi).wait()
            wait_gath(ss, j, rows0, semg0)
            scat(ds_, j, rows0)
            if s + 1 < _NSLAB:
                gath(ns, 0, rows0, semg0)
            wait_gath(ss, j + 1, rows1, semg1)
            scat(ds_, j + 1, rows1)

        plsc.subcore_barrier()

        pltpu.sync_copy(acc.at[pl.ds(sid * 640, 640)],
                        out_hbm.at[cid, pl.ds(sid * 640, 640)])

    return seg_kernel(h, src, dst, zrows)


def _bn_affine(st_ref, g_ref, be_ref):
    mean = st_ref[0:1, :] * (1.0 / _N)
    var = st_ref[1:2, :] * (1.0 / _N) - mean * mean
    scale = g_ref[...] * lax.rsqrt(var + 1e-5)
    shift = be_ref[...] - mean * scale
    return scale, shift


def _layer_body(h_ref, a_ref, w1_ref, b1_ref, g1_ref, be1_ref,
                w2_ref, b2_ref, g2_ref, be2_ref, ho_ref, z1_s, z2_s,
                st1, st2):
    p = pl.program_id(0)
    i = pl.program_id(1)

    @pl.when(jnp.logical_and(p == 0, i == 0))
    def _():
        st1[...] = jnp.zeros_like(st1)
        st2[...] = jnp.zeros_like(st2)

    @pl.when(p == 0)
    def _():
        u = h_ref[...] + a_ref[0] + a_ref[1]
        z = jnp.dot(u, w1_ref[...],
                    preferred_element_type=jnp.float32) + b1_ref[...]
        z1_s[pl.ds(i * _BLK, _BLK), :] = z
        st1[0:1, :] += jnp.sum(z, axis=0, keepdims=True)
        st1[1:2, :] += jnp.sum(z * z, axis=0, keepdims=True)

    @pl.when(p == 1)
    def _():
        scale, shift = _bn_affine(st1, g1_ref, be1_ref)
        a = jnp.maximum(z1_s[pl.ds(i * _BLK, _BLK), :] * scale + shift, 0.0)
        z2 = jnp.dot(a, w2_ref[...],
                     preferred_element_type=jnp.float32) + b2_ref[...]
        z2_s[pl.ds(i * _BLK, _BLK), :] = z2
        st2[0:1, :] += jnp.sum(z2, axis=0, keepdims=True)
        st2[1:2, :] += jnp.sum(z2 * z2, axis=0, keepdims=True)

    @pl.when(p == 2)
    def _():
        scale, shift = _bn_affine(st2, g2_ref, be2_ref)
        ho_ref[...] = jnp.maximum(
            z2_s[pl.ds(i * _BLK, _BLK), :] * scale + shift, 0.0)


def _heads_body(x_ref, h1_ref, h2_ref, h3_ref, fw_ref, fb_ref, o_ref):
    logits = jnp.dot(x_ref[...], fw_ref[0], preferred_element_type=jnp.float32)
    logits += jnp.dot(h1_ref[...], fw_ref[1], preferred_element_type=jnp.float32)
    logits += jnp.dot(h2_ref[...], fw_ref[2], preferred_element_type=jnp.float32)
    logits += jnp.dot(h3_ref[...], fw_ref[3], preferred_element_type=jnp.float32)
    logits += jnp.sum(fb_ref[...], axis=0, keepdims=True)
    m = jnp.max(logits, axis=-1, keepdims=True)
    e = jnp.exp(logits - m)
    o_ref[...] = logits - m - jnp.log(jnp.sum(e, axis=-1, keepdims=True))


def _row_spec():
    return pl.BlockSpec((_BLK, _D), lambda i: (i, 0))


def _full_spec(shape, ng=1):
    nd = len(shape)
    return pl.BlockSpec(shape, lambda *g: (0,) * nd)


def _gin_layer(h, agg2, w1, b1, g1, be1, w2, b2, g2, be2):
    p0row = pl.BlockSpec((_BLK, _D),
                         lambda p, i: (jnp.where(p == 0, i, 0), 0))
    return pl.pallas_call(
        _layer_body,
        grid=(3, _N // _BLK),
        in_specs=[
            p0row,
            pl.BlockSpec((_NC, _BLK, _D),
                         lambda p, i: (0, jnp.where(p == 0, i, 0), 0)),
            _full_spec((_D, _D)),
            _full_spec((1, _D)),
            _full_spec((1, _D)),
            _full_spec((1, _D)),
            _full_spec((_D, _D)),
            _full_spec((1, _D)),
            _full_spec((1, _D)),
            _full_spec((1, _D)),
        ],
        out_specs=pl.BlockSpec((_BLK, _D),
                               lambda p, i: (jnp.where(p == 2, i, 0), 0)),
        out_shape=jax.ShapeDtypeStruct((_N, _D), jnp.float32),
        scratch_shapes=[
            pltpu.VMEM((_N, _D), jnp.float32),
            pltpu.VMEM((_N, _D), jnp.float32),
            pltpu.VMEM((8, _D), jnp.float32),
            pltpu.VMEM((8, _D), jnp.float32),
        ],
    )(h, agg2, w1, b1, g1, be1, w2, b2, g2, be2)


def _heads(x, h1, h2, h3, fw, fb):
    return pl.pallas_call(
        _heads_body,
        grid=(_N // _BLK,),
        in_specs=[
            _row_spec(),
            _row_spec(),
            _row_spec(),
            _row_spec(),
            _full_spec((_L + 1, _D, _C)),
            _full_spec((_L + 1, _C)),
        ],
        out_specs=pl.BlockSpec((_BLK, _C), lambda i: (i, 0)),
        out_shape=jax.ShapeDtypeStruct((_N, _C), jnp.float32),
    )(x, h1, h2, h3, fw, fb)


def kernel(x, edge_index, cw1, cb1, cg1, cbe1, cw2, cb2, bng, bnb, fcw, fcb):
    npad = _EPAD - _E
    pad_i = jnp.arange(npad, dtype=jnp.int32)
    src = jnp.concatenate([edge_index[0], (pad_i * 37) % _N]).reshape(
        _NW, _NSLAB, _SLAB, _CH)
    dst = jnp.concatenate([edge_index[1], _N + pad_i % (_NPAD - _N)]).reshape(
        _NW, _NSLAB, _SLAB, _CH)
    zrows = jnp.zeros((640, _D), jnp.float32)

    h = x
    outs = [x]
    for l in range(_L):
        agg2 = _seg_sum_sc(h, src, dst, zrows)
        h = _gin_layer(h, agg2, cw1[l], cb1[l].reshape(1, _D),
                       cg1[l].reshape(1, _D), cbe1[l].reshape(1, _D),
                       cw2[l], cb2[l].reshape(1, _D),
                       bng[l].reshape(1, _D), bnb[l].reshape(1, _D))
        outs.append(h)

    out = _heads(outs[0], outs[1], outs[2], outs[3], fcw, fcb)
    return (out, 0)


# speedup vs baseline: 1.3364x; 1.0037x over previous
import functools

import numpy as np

import jax
import jax.numpy as jnp
from jax import lax
from jax.experimental import pallas as pl
from jax.experimental.pallas import tpu as pltpu
from jax.experimental.pallas import tpu_sc as plsc

_N = 10000
_E = 320000
_D = 128
_C = 16
_L = 3

_NPAD = 10240
_CH = 128
_NC = 2
_NS = 16
_NW = _NC * _NS
_SLAB = 16
_NSLAB = 5
_CPW = _SLAB * _NSLAB
_EPAD = _NW * _CPW * _CH

_BLK = 5000

_PAD_SRC = np.asarray((np.arange(_EPAD - _E) * 37) % _N, np.int32)
_PAD_DST = np.asarray(_N + np.arange(_EPAD - _E) % (_NPAD - _N), np.int32)


def _seg_sum_sc(h, src, dst, zrows):
    mesh = plsc.VectorSubcoreMesh(core_axis_name="c", subcore_axis_name="s")

    @functools.partial(
        pl.kernel,
        out_type=jax.ShapeDtypeStruct((_NC, _NPAD, _D), jnp.float32),
        mesh=mesh,
        scratch_types=[
            pltpu.VMEM((_SLAB, _CH), jnp.int32),
            pltpu.VMEM((_SLAB, _CH), jnp.int32),
            pltpu.VMEM((_SLAB, _CH), jnp.int32),
            pltpu.VMEM((_SLAB, _CH), jnp.int32),
            pltpu.VMEM((_CH, _D), jnp.float32),
            pltpu.VMEM((_CH, _D), jnp.float32),
            pltpu.VMEM_SHARED((_NPAD, _D), jnp.float32),
            pltpu.SemaphoreType.DMA,
            pltpu.SemaphoreType.DMA,
            pltpu.SemaphoreType.DMA,
        ],
    )
    def seg_kernel(h_hbm, src_hbm, dst_hbm, z_hbm, out_hbm,
                   src_s0, src_s1, dst_s0, dst_s1, rows0, rows1, acc,
                   semg0, semg1, semi):
        cid = lax.axis_index("c")
        sid = lax.axis_index("s")
        wid = sid * _NC + cid

        def gath(ss, j, rows, sem):
            pltpu.async_copy(h_hbm.at[ss.at[j]], rows, sem)

        def wait_gath(ss, j, rows, sem):
            pltpu.make_async_copy(h_hbm.at[ss.at[j]], rows, sem).wait()

        def scat(ds_, j, rows):
            pltpu.sync_copy(rows, acc.at[ds_.at[j]], add=True)

        pltpu.async_copy(src_hbm.at[wid, 0], src_s0, semi)
        pltpu.async_copy(dst_hbm.at[wid, 0], dst_s0, semi)
        pltpu.sync_copy(z_hbm, acc.at[pl.ds(sid * 640, 640)])
        pltpu.make_async_copy(src_hbm.at[wid, 0], src_s0, semi).wait()
        pltpu.make_async_copy(dst_hbm.at[wid, 0], dst_s0, semi).wait()
        plsc.subcore_barrier()
        gath(src_s0, 0, rows0, semg0)

        idx_bufs = [(src_s0, dst_s0), (src_s1, dst_s1)]
        npair = _SLAB // 2
        for s in range(_NSLAB):
            ss, ds_ = idx_bufs[s % 2]
            ns, nd = idx_bufs[(s + 1) % 2]
            if s + 1 < _NSLAB:
                pltpu.async_copy(src_hbm.at[wid, s + 1], ns, semi)
                pltpu.async_copy(dst_hbm.at[wid, s + 1], nd, semi)

            def body(g, carry, ss=ss, ds_=ds_):
                j = 2 * g
                gath(ss, j + 1, rows1, semg1)
                wait_gath(ss, j, rows0, semg0)
                scat(ds_, j, rows0)
                gath(ss, j + 2, rows0, semg0)
                wait_gath(ss, j + 1, rows1, semg1)
                scat(ds_, j + 1, rows1)
                return carry

            lax.fori_loop(0, npair - 1, body, 0)

            j = _SLAB - 2
            gath(ss, j + 1, rows1, semg1)
            if s + 1 < _NSLAB:
                pltpu.make_async_copy(src_hbm.at[wid, s + 1], ns,
                                      semi).wait()
                pltpu.make_async_copy(dst_hbm.at[wid, s + 1], nd,
                                      semi).wait()
            wait_gath(ss, j, rows0, semg0)
            scat(ds_, j, rows0)
            if s + 1 < _NSLAB:
                gath(ns, 0, rows0, semg0)
            wait_gath(ss, j + 1, rows1, semg1)
            scat(ds_, j + 1, rows1)

        plsc.subcore_barrier()

        pltpu.sync_copy(acc.at[pl.ds(sid * 640, 640)],
                        out_hbm.at[cid, pl.ds(sid * 640, 640)])

    return seg_kernel(h, src, dst, zrows)


def _bn_affine(st_ref, g_ref, be_ref):
    mean = st_ref[0:1, :] * (1.0 / _N)
    var = st_ref[1:2, :] * (1.0 / _N) - mean * mean
    scale = g_ref[...] * lax.rsqrt(var + 1e-5)
    shift = be_ref[...] - mean * scale
    return scale, shift


def _layer_body(h_ref, a_ref, w1_ref, b1_ref, g1_ref, be1_ref,
                w2_ref, b2_ref, g2_ref, be2_ref, ho_ref, z1_s, z2_s,
                st1, st2):
    p = pl.program_id(0)
    i = pl.program_id(1)

    @pl.when(jnp.logical_and(p == 0, i == 0))
    def _():
        st1[...] = jnp.zeros_like(st1)
        st2[...] = jnp.zeros_like(st2)

    @pl.when(p == 0)
    def _():
        u = h_ref[...] + a_ref[0] + a_ref[1]
        z = jnp.dot(u, w1_ref[...],
                    preferred_element_type=jnp.float32) + b1_ref[...]
        z1_s[pl.ds(i * _BLK, _BLK), :] = z
        st1[0:1, :] += jnp.sum(z, axis=0, keepdims=True)
        st1[1:2, :] += jnp.sum(z * z, axis=0, keepdims=True)

    @pl.when(p == 1)
    def _():
        scale, shift = _bn_affine(st1, g1_ref, be1_ref)
        a = jnp.maximum(z1_s[pl.ds(i * _BLK, _BLK), :] * scale + shift, 0.0)
        z2 = jnp.dot(a, w2_ref[...],
                     preferred_element_type=jnp.float32) + b2_ref[...]
        z2_s[pl.ds(i * _BLK, _BLK), :] = z2
        st2[0:1, :] += jnp.sum(z2, axis=0, keepdims=True)
        st2[1:2, :] += jnp.sum(z2 * z2, axis=0, keepdims=True)

    @pl.when(p == 2)
    def _():
        scale, shift = _bn_affine(st2, g2_ref, be2_ref)
        ho_ref[...] = jnp.maximum(
            z2_s[pl.ds(i * _BLK, _BLK), :] * scale + shift, 0.0)


def _heads_body(x_ref, h1_ref, h2_ref, h3_ref, fw_ref, fb_ref, o_ref):
    logits = jnp.dot(x_ref[...], fw_ref[0], preferred_element_type=jnp.float32)
    logits += jnp.dot(h1_ref[...], fw_ref[1], preferred_element_type=jnp.float32)
    logits += jnp.dot(h2_ref[...], fw_ref[2], preferred_element_type=jnp.float32)
    logits += jnp.dot(h3_ref[...], fw_ref[3], preferred_element_type=jnp.float32)
    logits += jnp.sum(fb_ref[...], axis=0, keepdims=True)
    m = jnp.max(logits, axis=-1, keepdims=True)
    e = jnp.exp(logits - m)
    o_ref[...] = logits - m - jnp.log(jnp.sum(e, axis=-1, keepdims=True))


def _row_spec():
    return pl.BlockSpec((_BLK, _D), lambda i: (i, 0))


def _full_spec(shape, ng=1):
    nd = len(shape)
    return pl.BlockSpec(shape, lambda *g: (0,) * nd)


def _gin_layer(h, agg2, w1, b1, g1, be1, w2, b2, g2, be2):
    p0row = pl.BlockSpec((_BLK, _D),
                         lambda p, i: (jnp.where(p == 0, i, 0), 0))
    return pl.pallas_call(
        _layer_body,
        grid=(3, _N // _BLK),
        in_specs=[
            p0row,
            pl.BlockSpec((_NC, _BLK, _D),
                         lambda p, i: (0, jnp.where(p == 0, i, 0), 0)),
            _full_spec((_D, _D)),
            _full_spec((1, _D)),
            _full_spec((1, _D)),
            _full_spec((1, _D)),
            _full_spec((_D, _D)),
            _full_spec((1, _D)),
            _full_spec((1, _D)),
            _full_spec((1, _D)),
        ],
        out_specs=pl.BlockSpec((_BLK, _D),
                               lambda p, i: (jnp.where(p == 2, i, 0), 0)),
        out_shape=jax.ShapeDtypeStruct((_N, _D), jnp.float32),
        scratch_shapes=[
            pltpu.VMEM((_N, _D), jnp.float32),
            pltpu.VMEM((_N, _D), jnp.float32),
            pltpu.VMEM((8, _D), jnp.float32),
            pltpu.VMEM((8, _D), jnp.float32),
        ],
    )(h, agg2, w1, b1, g1, be1, w2, b2, g2, be2)


def _heads(x, h1, h2, h3, fw, fb):
    return pl.pallas_call(
        _heads_body,
        grid=(_N // _BLK,),
        in_specs=[
            _row_spec(),
            _row_spec(),
            _row_spec(),
            _row_spec(),
            _full_spec((_L + 1, _D, _C)),
            _full_spec((_L + 1, _C)),
        ],
        out_specs=pl.BlockSpec((_BLK, _C), lambda i: (i, 0)),
        out_shape=jax.ShapeDtypeStruct((_N, _C), jnp.float32),
    )(x, h1, h2, h3, fw, fb)


def kernel(x, edge_index, cw1, cb1, cg1, cbe1, cw2, cb2, bng, bnb, fcw, fcb):
    src = jnp.concatenate([edge_index[0], _PAD_SRC]).reshape(
        _NW, _NSLAB, _SLAB, _CH)
    dst = jnp.concatenate([edge_index[1], _PAD_DST]).reshape(
        _NW, _NSLAB, _SLAB, _CH)
    zrows = jnp.zeros((640, _D), jnp.float32)

    h = x
    outs = [x]
    for l in range(_L):
        agg2 = _seg_sum_sc(h, src, dst, zrows)
        h = _gin_layer(h, agg2, cw1[l], cb1[l].reshape(1, _D),
                       cg1[l].reshape(1, _D), cbe1[l].reshape(1, _D),
                       cw2[l], cb2[l].reshape(1, _D),
                       bng[l].reshape(1, _D), bnb[l].reshape(1, _D))
        outs.append(h)

    out = _heads(outs[0], outs[1], outs[2], outs[3], fcw, fcb)
    return (out, 0)
